# Initial kernel scaffold; baseline (speedup 1.0000x reference)
import functools

import jax
import jax.numpy as jnp
from jax import lax
from jax.experimental import pallas as pl
from jax.experimental.pallas import tpu as pltpu
from jax.experimental.pallas import tpu_sc as plsc

NC = 2
NS = 16
NW = NC * NS
GCHUNK = 128
LANES = 16


@functools.partial(jax.jit, static_argnums=(3, 4, 5, 6))
def _cp_forward(flat_idx, table, weights, B, H, V, R):
    BW = B // NW
    RW = BW * H
    C = 64
    NCH = BW // C
    CR = C * H
    ND = CR // GCHUNK
    mesh = plsc.VectorSubcoreMesh(core_axis_name="c", subcore_axis_name="s")

    @functools.partial(
        pl.kernel,
        out_type=jax.ShapeDtypeStruct((B,), jnp.float32),
        mesh=mesh,
        scratch_types=[
            pltpu.VMEM((RW // GCHUNK, GCHUNK), jnp.int32),
            pltpu.VMEM((CR, R), jnp.float32),
            pltpu.VMEM((CR, R), jnp.float32),
            pltpu.VMEM((BW,), jnp.float32),
            pltpu.VMEM((R,), jnp.float32),
            pltpu.SemaphoreType.DMA,
            pltpu.SemaphoreType.DMA,
        ],
    )
    def k(idx_hbm, table_hbm, w_hbm, out_hbm,
          idx_v, rows_a, rows_b, out_v, w_v, sem_a, sem_b):
        wid = lax.axis_index("s") * NC + lax.axis_index("c")
        pltpu.sync_copy(idx_hbm.at[wid], idx_v)
        pltpu.sync_copy(w_hbm, w_v)
        bufs = (rows_a, rows_b)
        sems = (sem_a, sem_b)

        def issue(c):
            buf, sem = bufs[c % 2], sems[c % 2]
            for j in range(ND):
                pltpu.async_copy(table_hbm.at[idx_v.at[c * ND + j]],
                                 buf.at[pl.ds(j * GCHUNK, GCHUNK)], sem)

        def drain(c):
            buf, sem = bufs[c % 2], sems[c % 2]
            for j in range(ND):
                pltpu.make_async_copy(table_hbm.at[idx_v.at[c * ND + j]],
                                      buf.at[pl.ds(j * GCHUNK, GCHUNK)],
                                      sem).wait()

        w_lo = w_v[0:LANES]
        w_hi = w_v[LANES:2 * LANES]

        def compute(c):
            buf = bufs[c % 2]

            def body(b, _):
                rbase = b * H
                acc_lo = buf[rbase, 0:LANES]
                acc_hi = buf[rbase, LANES:2 * LANES]
                for m in range(1, H):
                    acc_lo = acc_lo * buf[rbase + m, 0:LANES]
                    acc_hi = acc_hi * buf[rbase + m, LANES:2 * LANES]
                t = acc_lo * w_lo + acc_hi * w_hi
                out_v[c * C + b] = jnp.sum(t)
                return None

            lax.fori_loop(0, C, body, None)

        issue(0)
        for c in range(NCH):
            if c + 1 < NCH:
                issue(c + 1)
            drain(c)
            compute(c)
        pltpu.sync_copy(out_v, out_hbm.at[pl.ds(wid * BW, BW)])

    return k(flat_idx, table, weights)


def kernel(coords, factors, weights):
    H, V, R = factors.shape
    B = coords.shape[0]
    coords32 = coords.astype(jnp.int32)
    offs = (jnp.arange(H, dtype=jnp.int32) * V)[None, :]
    flat_idx = (coords32 + offs).reshape(NW, (B * H) // (NW * GCHUNK), GCHUNK)
    table = factors.reshape(H * V, R)
    return _cp_forward(flat_idx, table, weights.astype(jnp.float32),
                       B, H, V, R)


# speedup vs baseline: 1.1595x; 1.1595x over previous
import functools

import numpy as np

import jax
import jax.numpy as jnp
from jax import lax
from jax.experimental import pallas as pl
from jax.experimental.pallas import tpu as pltpu
from jax.experimental.pallas import tpu_sc as plsc

NC = 2
NS = 16
NW = NC * NS
GCHUNK = 128
LANES = 16


@functools.partial(jax.jit, static_argnums=(3, 4, 5, 6))
def _cp_forward(flat_idx, table, weights, B, H, V, R):
    BW = B // NW
    RW = BW * H
    C = 64
    NCH = BW // C
    CR = C * H
    ND = CR // GCHUNK
    mesh = plsc.VectorSubcoreMesh(core_axis_name="c", subcore_axis_name="s")

    @functools.partial(
        pl.kernel,
        out_type=jax.ShapeDtypeStruct((B,), jnp.float32),
        mesh=mesh,
        scratch_types=[
            pltpu.VMEM((RW // GCHUNK, GCHUNK), jnp.int32),
            pltpu.VMEM((CR, R), jnp.float32),
            pltpu.VMEM((CR, R), jnp.float32),
            pltpu.VMEM((BW,), jnp.float32),
            pltpu.VMEM((R,), jnp.float32),
            pltpu.SemaphoreType.DMA,
            pltpu.SemaphoreType.DMA,
        ],
        compiler_params=pltpu.CompilerParams(needs_layout_passes=False,
                                             use_tc_tiling_on_sc=False),
    )
    def k(idx_hbm, table_hbm, w_hbm, out_hbm,
          idx_v, rows_a, rows_b, out_v, w_v, sem_a, sem_b):
        def i32(x):
            return lax.convert_element_type(x, jnp.int32)

        wid = i32(lax.axis_index("s") * NC + lax.axis_index("c"))
        pltpu.sync_copy(idx_hbm.at[wid], idx_v)
        pltpu.sync_copy(w_hbm, w_v)
        bufs = (rows_a, rows_b)
        sems = (sem_a, sem_b)

        def issue(c, par):
            buf, sem = bufs[par], sems[par]
            for j in range(ND):
                pltpu.async_copy(table_hbm.at[idx_v.at[i32(c * ND + j)]],
                                 buf.at[pl.ds(j * GCHUNK, GCHUNK)], sem)

        def drain(c, par):
            buf, sem = bufs[par], sems[par]
            for j in range(ND):
                pltpu.make_async_copy(table_hbm.at[idx_v.at[i32(c * ND + j)]],
                                      buf.at[pl.ds(j * GCHUNK, GCHUNK)],
                                      sem).wait()

        w_lo = w_v[0:LANES]
        w_hi = w_v[LANES:2 * LANES]
        lane = lax.iota(jnp.int32, LANES)

        def compute(c, par):
            buf = bufs[par]

            def body(b, vec):
                b = i32(b)
                pos = b & (LANES - 1)
                rbase = b * H
                acc_lo = buf[rbase, 0:LANES]
                acc_hi = buf[rbase, LANES:2 * LANES]
                for m in range(1, H):
                    acc_lo = acc_lo * buf[rbase + m, 0:LANES]
                    acc_hi = acc_hi * buf[rbase + m, LANES:2 * LANES]
                t = acc_lo * w_lo + acc_hi * w_hi
                vec = jnp.where(lane == pos, jnp.sum(t), vec)
                full = pos == LANES - 1

                @pl.when(full)
                def _():
                    out_v[pl.ds(i32(c * C + b - (LANES - 1)), LANES)] = vec

                return vec * jnp.where(full, 0.0, 1.0).astype(jnp.float32)

            lax.fori_loop(np.int32(0), np.int32(C), body,
                          jnp.zeros((LANES,), jnp.float32))

        issue(0, 0)

        @pl.loop(np.int32(0), np.int32(NCH), step=np.int32(2))
        def _(cc):
            for par in range(2):
                c = i32(cc) + par

                @pl.when(c + 1 < NCH)
                def _():
                    issue(c + 1, (par + 1) % 2)

                drain(c, par)
                compute(c, par)

        pltpu.sync_copy(out_v, out_hbm.at[pl.ds(wid * BW, BW)])

    with jax.enable_x64(False):
        return k(flat_idx, table, weights)


def kernel(coords, factors, weights):
    H, V, R = factors.shape
    B = coords.shape[0]
    coords32 = coords.astype(jnp.int32)
    offs = (jnp.arange(H, dtype=jnp.int32) * V)[None, :]
    flat_idx = (coords32 + offs).reshape(NW, (B * H) // (NW * GCHUNK), GCHUNK)
    table = factors.reshape(H * V, R)
    return _cp_forward(flat_idx, table, weights.astype(jnp.float32),
                       B, H, V, R)


# speedup vs baseline: 2.9690x; 2.5607x over previous
import functools

import numpy as np

import jax
import jax.numpy as jnp
from jax import lax
from jax.experimental import pallas as pl
from jax.experimental.pallas import tpu as pltpu
from jax.experimental.pallas import tpu_sc as plsc

NC = 2
NS = 16
LANES = 16


@functools.partial(jax.jit, static_argnums=(3, 4, 5, 6))
def _cp_partials(coords_t, table_t, weights, B, H, V, R):
    assert R == NC * NS
    QB = 4096
    NQ = B // QB
    mesh = plsc.VectorSubcoreMesh(core_axis_name="c", subcore_axis_name="s")

    @functools.partial(
        pl.kernel,
        out_type=jax.ShapeDtypeStruct((R, B), jnp.float32),
        mesh=mesh,
        scratch_types=[
            pltpu.VMEM((V,), jnp.float32),
            pltpu.VMEM((B,), jnp.float32),
            pltpu.VMEM((QB,), jnp.int32),
            pltpu.VMEM((R,), jnp.float32),
        ],
        compiler_params=pltpu.CompilerParams(needs_layout_passes=False),
    )
    def k(ct_hbm, tab_hbm, w_hbm, p_hbm,
          row_v, prod_v, cq_v, w_v):
        def i32(x):
            return lax.convert_element_type(x, jnp.int32)

        c = i32(lax.axis_index("c"))
        s = i32(lax.axis_index("s"))
        r = c * NS + s
        pltpu.sync_copy(w_hbm, w_v)
        w_bc = plsc.load_gather(w_v, [jnp.full((LANES,), r, jnp.int32)])

        def gather_pass(m, first):
            pltpu.sync_copy(tab_hbm.at[m, r], row_v)
            for q in range(NQ):
                pltpu.sync_copy(ct_hbm.at[m, pl.ds(np.int32(q * QB), QB)],
                                cq_v)

                def body(iv, _):
                    iv = i32(iv)
                    off = iv * LANES
                    idx = cq_v[pl.ds(off, LANES)]
                    vals = plsc.load_gather(row_v, [idx])
                    pslice = pl.ds(np.int32(q * QB) + off, LANES)
                    if first:
                        prod_v[pslice] = vals * w_bc
                    else:
                        prod_v[pslice] = prod_v[pslice] * vals
                    return None

                lax.fori_loop(np.int32(0), np.int32(QB // LANES), body, None)

        gather_pass(np.int32(0), True)

        @pl.loop(np.int32(1), np.int32(H))
        def _(m):
            gather_pass(i32(m), False)

        pltpu.sync_copy(prod_v, p_hbm.at[r])

    return k(coords_t, table_t, weights)


def _combine(p):
    def k2(p_ref, o_ref):
        o_ref[...] = jnp.sum(p_ref[...], axis=0)

    return pl.pallas_call(
        k2,
        out_shape=jax.ShapeDtypeStruct((p.shape[1],), jnp.float32),
    )(p)


def kernel(coords, factors, weights):
    H, V, R = factors.shape
    B = coords.shape[0]
    coords_t = coords.astype(jnp.int32).T
    table_t = jnp.transpose(factors, (0, 2, 1))
    with jax.enable_x64(False):
        p = _cp_partials(coords_t, table_t, weights.astype(jnp.float32),
                         B, H, V, R)
        return _combine(p)
